# speedup vs baseline: 1.2832x; 1.2832x over previous
import jax
import jax.numpy as jnp
from jax.experimental import pallas as pl
from jax.experimental.pallas import tpu as pltpu

B = 8
NPER = 1024
M1 = 512
M2 = 128
K = 64
R1 = 0.2
R2 = 0.4
N_ACTIONS = 12


def _fps(pos_b, m):
    d0 = jnp.sum((pos_b - pos_b[:, 0:1, :]) ** 2, axis=-1)
    idxs = jnp.zeros((pos_b.shape[0], m), dtype=jnp.int32)

    def body(i, state):
        idxs, mind = state
        nxt = jnp.argmax(mind, axis=1).astype(jnp.int32)
        idxs = idxs.at[:, i].set(nxt)
        p = jnp.take_along_axis(pos_b, nxt[:, None, None], axis=1)
        d = jnp.sum((pos_b - p) ** 2, axis=-1)
        mind = jnp.minimum(mind, d)
        return idxs, mind

    idxs, _ = jax.lax.fori_loop(1, m, body, (idxs, d0))
    return idxs


def _radius(pos_b, qpos, r):
    d2 = jnp.sum((qpos[:, :, None, :] - pos_b[:, None, :, :]) ** 2, axis=-1)
    negv, nbr = jax.lax.top_k(-d2, K)
    mask = (-negv) <= r * r + 1e-12
    return nbr, mask


def _gather(xb, nbr):
    Bb, m, k = nbr.shape
    out = jnp.take_along_axis(xb, nbr.reshape(Bb, m * k)[:, :, None], axis=1)
    return out.reshape(Bb, m, k, xb.shape[-1])


def _take(xb, idx):
    return jnp.take_along_axis(xb, idx[:, :, None], axis=1)


def _mlp_bn(x, layers, mask=None):
    for lyr in layers[:-1]:
        W, b, g, be = lyr
        x = x @ W + b
        if mask is None:
            mu = jnp.mean(x, axis=0)
            var = jnp.mean((x - mu) ** 2, axis=0)
        else:
            w = mask / jnp.maximum(jnp.sum(mask), 1.0)
            mu = jnp.sum(w[:, None] * x, axis=0)
            var = jnp.sum(w[:, None] * (x - mu) ** 2, axis=0)
        x = (x - mu) / jnp.sqrt(var + 1e-5) * g + be
        x = jax.nn.relu(x)
    W, b = layers[-1]
    return x @ W + b


def _point_conv(x_b, pos_b, qpos, nbr, mask, layers):
    h = _gather(pos_b, nbr) - qpos[:, :, None, :]
    if x_b is not None:
        h = jnp.concatenate([_gather(x_b, nbr), h], axis=-1)
    Bb, m, k, F = h.shape
    out = _mlp_bn(h.reshape(Bb * m * k, F), layers, mask.reshape(Bb * m * k).astype(jnp.float32))
    out = out.reshape(Bb, m, k, -1)
    out = jnp.where(mask[..., None], out, -jnp.inf)
    return jnp.max(out, axis=2)


def _tail_kernel(h_ref, w1, b1, g1, be1, w2, b2, g2, be2, w3, b3,
                 pw1, pb1, pw2, pb2, pw3, pb3,
                 vw1, vb1, vw2, vb2, vw3, vb3,
                 probs_ref, value_ref):
    x = h_ref[...]
    x = jnp.dot(x, w1[...], preferred_element_type=jnp.float32) + b1[...]
    mu = jnp.mean(x, axis=0)
    var = jnp.mean((x - mu) ** 2, axis=0)
    x = (x - mu) / jnp.sqrt(var + 1e-5) * g1[...] + be1[...]
    x = jax.nn.relu(x)
    x = jnp.dot(x, w2[...], preferred_element_type=jnp.float32) + b2[...]
    mu = jnp.mean(x, axis=0)
    var = jnp.mean((x - mu) ** 2, axis=0)
    x = (x - mu) / jnp.sqrt(var + 1e-5) * g2[...] + be2[...]
    x = jax.nn.relu(x)
    x = jnp.dot(x, w3[...], preferred_element_type=jnp.float32) + b3[...]
    g = jnp.max(x.reshape(B, M2, x.shape[-1]), axis=1)
    p = jax.nn.relu(jnp.dot(g, pw1[...], preferred_element_type=jnp.float32) + pb1[...])
    p = jax.nn.relu(jnp.dot(p, pw2[...], preferred_element_type=jnp.float32) + pb2[...])
    logits = jnp.dot(p, pw3[...], preferred_element_type=jnp.float32) + pb3[...]
    probs_ref[...] = jax.nn.softmax(logits, axis=-1)
    v = jax.nn.relu(jnp.dot(g, vw1[...], preferred_element_type=jnp.float32) + vb1[...])
    v = jax.nn.relu(jnp.dot(v, vw2[...], preferred_element_type=jnp.float32) + vb2[...])
    value_ref[...] = jnp.dot(v, vw3[...], preferred_element_type=jnp.float32) + vb3[...]


def _dense_tail(h, params):
    (w1, b1, g1, be1), (w2, b2, g2, be2), (w3, b3) = params["nn3"]
    (pw1, pb1), (pw2, pb2), (pw3, pb3) = params["pi"]
    (vw1, vb1), (vw2, vb2), (vw3, vb3) = params["value"]
    probs, value = pl.pallas_call(
        _tail_kernel,
        out_shape=(
            jax.ShapeDtypeStruct((B, N_ACTIONS), jnp.float32),
            jax.ShapeDtypeStruct((B, 1), jnp.float32),
        ),
    )(h, w1, b1, g1, be1, w2, b2, g2, be2, w3, b3,
      pw1, pb1, pw2, pb2, pw3, pb3,
      vw1, vb1, vw2, vb2, vw3, vb3)
    return probs, value[:, 0]


def kernel(pos, ptr, params):
    Bb = ptr.shape[0] - 1
    n = pos.shape[0] // Bb
    pos_b = pos.reshape(Bb, n, 3)
    pos_b = jax.lax.stop_gradient(pos_b)
    q1 = _fps(pos_b, M1)
    pos1 = _take(pos_b, q1)
    nbr1, mask1 = _radius(pos_b, pos1, R1)
    q2 = _fps(pos1, M2)
    pos2 = _take(pos1, q2)
    nbr2, mask2 = _radius(pos1, pos2, R2)

    x1 = _point_conv(None, pos_b, pos1, nbr1, mask1, params["nn1"])
    x2 = _point_conv(x1, pos1, pos2, nbr2, mask2, params["nn2"])
    h = jnp.concatenate([x2, pos2], axis=-1).reshape(Bb * M2, -1)
    return _dense_tail(h, params)


# speedup vs baseline: 1.6900x; 1.3170x over previous
import jax
import jax.numpy as jnp
from jax.experimental import pallas as pl
from jax.experimental.pallas import tpu as pltpu

B = 8
NPER = 1024
M1 = 512
M2 = 128
K = 64
R1 = 0.2
R2 = 0.4
N_ACTIONS = 12


def _argmax_lanes(v):
    n = v.shape[1]
    mx = jnp.max(v, axis=1, keepdims=True)
    iota = jax.lax.broadcasted_iota(jnp.int32, v.shape, 1)
    return jnp.min(jnp.where(v == mx, iota, n), axis=1).astype(jnp.int32)


def _onehot_pick(v, nxt):
    iota = jax.lax.broadcasted_iota(jnp.int32, v.shape, 1)
    return jnp.sum(jnp.where(iota == nxt[:, None], v, 0.0), axis=1, keepdims=True)


def _fps_loop(xs, ys, zs, m):
    x0 = xs[:, 0:1]
    y0 = ys[:, 0:1]
    z0 = zs[:, 0:1]
    d0 = (xs - x0) * (xs - x0) + (ys - y0) * (ys - y0) + (zs - z0) * (zs - z0)
    oiota = jax.lax.broadcasted_iota(jnp.int32, (B, m), 1)
    q0 = jnp.zeros((B, m), jnp.int32)
    px0 = jnp.broadcast_to(x0, (B, m))
    py0 = jnp.broadcast_to(y0, (B, m))
    pz0 = jnp.broadcast_to(z0, (B, m))

    def body(i, state):
        mind, q, pxs, pys, pzs = state
        nxt = _argmax_lanes(mind)
        px = _onehot_pick(xs, nxt)
        py = _onehot_pick(ys, nxt)
        pz = _onehot_pick(zs, nxt)
        sel = oiota == i
        q = jnp.where(sel, nxt[:, None], q)
        pxs = jnp.where(sel, px, pxs)
        pys = jnp.where(sel, py, pys)
        pzs = jnp.where(sel, pz, pzs)
        d = (xs - px) * (xs - px) + (ys - py) * (ys - py) + (zs - pz) * (zs - pz)
        return jnp.minimum(mind, d), q, pxs, pys, pzs

    _, q, pxs, pys, pzs = jax.lax.fori_loop(1, m, body, (d0, q0, px0, py0, pz0))
    return q, pxs, pys, pzs


def _fps_kernel(xs_ref, ys_ref, zs_ref,
                q1_ref, p1x_ref, p1y_ref, p1z_ref,
                q2_ref, p2x_ref, p2y_ref, p2z_ref):
    xs = xs_ref[...]
    ys = ys_ref[...]
    zs = zs_ref[...]
    q1, x1, y1, z1 = _fps_loop(xs, ys, zs, M1)
    q1_ref[...] = q1
    p1x_ref[...] = x1
    p1y_ref[...] = y1
    p1z_ref[...] = z1
    q2, x2, y2, z2 = _fps_loop(x1, y1, z1, M2)
    q2_ref[...] = q2
    p2x_ref[...] = x2
    p2y_ref[...] = y2
    p2z_ref[...] = z2


def _fps_pallas(xs, ys, zs):
    outs = pl.pallas_call(
        _fps_kernel,
        out_shape=(
            jax.ShapeDtypeStruct((B, M1), jnp.int32),
            jax.ShapeDtypeStruct((B, M1), jnp.float32),
            jax.ShapeDtypeStruct((B, M1), jnp.float32),
            jax.ShapeDtypeStruct((B, M1), jnp.float32),
            jax.ShapeDtypeStruct((B, M2), jnp.int32),
            jax.ShapeDtypeStruct((B, M2), jnp.float32),
            jax.ShapeDtypeStruct((B, M2), jnp.float32),
            jax.ShapeDtypeStruct((B, M2), jnp.float32),
        ),
    )(xs, ys, zs)
    return outs


def _radius(pos_b, qpos, r):
    d2 = jnp.sum((qpos[:, :, None, :] - pos_b[:, None, :, :]) ** 2, axis=-1)
    negv, nbr = jax.lax.top_k(-d2, K)
    mask = (-negv) <= r * r + 1e-12
    return nbr, mask


def _gather(xb, nbr):
    Bb, m, k = nbr.shape
    out = jnp.take_along_axis(xb, nbr.reshape(Bb, m * k)[:, :, None], axis=1)
    return out.reshape(Bb, m, k, xb.shape[-1])


def _take(xb, idx):
    return jnp.take_along_axis(xb, idx[:, :, None], axis=1)


def _mlp_bn(x, layers, mask=None):
    for lyr in layers[:-1]:
        W, b, g, be = lyr
        x = x @ W + b
        if mask is None:
            mu = jnp.mean(x, axis=0)
            var = jnp.mean((x - mu) ** 2, axis=0)
        else:
            w = mask / jnp.maximum(jnp.sum(mask), 1.0)
            mu = jnp.sum(w[:, None] * x, axis=0)
            var = jnp.sum(w[:, None] * (x - mu) ** 2, axis=0)
        x = (x - mu) / jnp.sqrt(var + 1e-5) * g + be
        x = jax.nn.relu(x)
    W, b = layers[-1]
    return x @ W + b


def _point_conv(x_b, pos_b, qpos, nbr, mask, layers):
    h = _gather(pos_b, nbr) - qpos[:, :, None, :]
    if x_b is not None:
        h = jnp.concatenate([_gather(x_b, nbr), h], axis=-1)
    Bb, m, k, F = h.shape
    out = _mlp_bn(h.reshape(Bb * m * k, F), layers, mask.reshape(Bb * m * k).astype(jnp.float32))
    out = out.reshape(Bb, m, k, -1)
    out = jnp.where(mask[..., None], out, -jnp.inf)
    return jnp.max(out, axis=2)


def _tail_kernel(h_ref, w1, b1, g1, be1, w2, b2, g2, be2, w3, b3,
                 pw1, pb1, pw2, pb2, pw3, pb3,
                 vw1, vb1, vw2, vb2, vw3, vb3,
                 probs_ref, value_ref):
    x = h_ref[...]
    x = jnp.dot(x, w1[...], preferred_element_type=jnp.float32) + b1[...]
    mu = jnp.mean(x, axis=0)
    var = jnp.mean((x - mu) ** 2, axis=0)
    x = (x - mu) / jnp.sqrt(var + 1e-5) * g1[...] + be1[...]
    x = jax.nn.relu(x)
    x = jnp.dot(x, w2[...], preferred_element_type=jnp.float32) + b2[...]
    mu = jnp.mean(x, axis=0)
    var = jnp.mean((x - mu) ** 2, axis=0)
    x = (x - mu) / jnp.sqrt(var + 1e-5) * g2[...] + be2[...]
    x = jax.nn.relu(x)
    x = jnp.dot(x, w3[...], preferred_element_type=jnp.float32) + b3[...]
    g = jnp.max(x.reshape(B, M2, x.shape[-1]), axis=1)
    p = jax.nn.relu(jnp.dot(g, pw1[...], preferred_element_type=jnp.float32) + pb1[...])
    p = jax.nn.relu(jnp.dot(p, pw2[...], preferred_element_type=jnp.float32) + pb2[...])
    logits = jnp.dot(p, pw3[...], preferred_element_type=jnp.float32) + pb3[...]
    probs_ref[...] = jax.nn.softmax(logits, axis=-1)
    v = jax.nn.relu(jnp.dot(g, vw1[...], preferred_element_type=jnp.float32) + vb1[...])
    v = jax.nn.relu(jnp.dot(v, vw2[...], preferred_element_type=jnp.float32) + vb2[...])
    value_ref[...] = jnp.dot(v, vw3[...], preferred_element_type=jnp.float32) + vb3[...]


def _dense_tail(h, params):
    (w1, b1, g1, be1), (w2, b2, g2, be2), (w3, b3) = params["nn3"]
    (pw1, pb1), (pw2, pb2), (pw3, pb3) = params["pi"]
    (vw1, vb1), (vw2, vb2), (vw3, vb3) = params["value"]
    probs, value = pl.pallas_call(
        _tail_kernel,
        out_shape=(
            jax.ShapeDtypeStruct((B, N_ACTIONS), jnp.float32),
            jax.ShapeDtypeStruct((B, 1), jnp.float32),
        ),
    )(h, w1, b1, g1, be1, w2, b2, g2, be2, w3, b3,
      pw1, pb1, pw2, pb2, pw3, pb3,
      vw1, vb1, vw2, vb2, vw3, vb3)
    return probs, value[:, 0]


def kernel(pos, ptr, params):
    Bb = ptr.shape[0] - 1
    n = pos.shape[0] // Bb
    pos_b = pos.reshape(Bb, n, 3)
    pos_b = jax.lax.stop_gradient(pos_b)
    xs = pos_b[:, :, 0]
    ys = pos_b[:, :, 1]
    zs = pos_b[:, :, 2]
    (q1, p1x, p1y, p1z, q2, p2x, p2y, p2z) = _fps_pallas(xs, ys, zs)
    pos1 = jnp.stack([p1x, p1y, p1z], axis=-1)
    pos2 = jnp.stack([p2x, p2y, p2z], axis=-1)
    nbr1, mask1 = _radius(pos_b, pos1, R1)
    nbr2, mask2 = _radius(pos1, pos2, R2)

    x1 = _point_conv(None, pos_b, pos1, nbr1, mask1, params["nn1"])
    x2 = _point_conv(x1, pos1, pos2, nbr2, mask2, params["nn2"])
    h = jnp.concatenate([x2, pos2], axis=-1).reshape(Bb * M2, -1)
    return _dense_tail(h, params)
